# baseline (speedup 1.0000x reference)
import functools

import jax
import jax.numpy as jnp
from jax import lax
from jax.experimental import pallas as pl
from jax.experimental.pallas import tpu as pltpu
from jax.experimental.pallas import tpu_sc as plsc

NODE_DIM = 128
EDGE_DIM = 16
OUT_DIM = NODE_DIM + EDGE_DIM
MAX_BOND = 8
NCOMBO = 512
LANES_PER_STEP = 256


def _sc_gather_call(am64, table, bondw):
    mesh = plsc.VectorSubcoreMesh(core_axis_name="c", subcore_axis_name="s")

    @functools.partial(
        pl.kernel,
        mesh=mesh,
        out_type=(
            jax.ShapeDtypeStruct((64, NODE_DIM), jnp.float32),
            jax.ShapeDtypeStruct((64, 128), jnp.int32),
        ),
        scratch_types=[
            pltpu.VMEM((64,), jnp.int32),
            pltpu.VMEM((64,), jnp.int32),
            pltpu.VMEM((64, NODE_DIM), jnp.float32),
            pltpu.VMEM((64, 128), jnp.int32),
            pltpu.SemaphoreType.DMA,
            pltpu.SemaphoreType.DMA,
        ],
    )
    def sc_gather(am_hbm, table_hbm, bondw_hbm, node_out, brow_out, idx_v, idx7_v, rows_v, brow_v, sem1, sem2):
        cid = lax.axis_index("c")
        sid = lax.axis_index("s")
        wid = sid * 2 + cid

        @pl.when(wid == 0)
        def _():
            pltpu.sync_copy(am_hbm, idx_v)
            for k in range(4):
                idx7_v[pl.ds(k * 16, 16)] = idx_v[pl.ds(k * 16, 16)] >> 7
            cp1 = pltpu.async_copy(table_hbm.at[idx_v], rows_v, sem1)
            cp2 = pltpu.async_copy(bondw_hbm.at[idx7_v], brow_v, sem2)
            cp1.wait()
            cp2.wait()
            pltpu.sync_copy(rows_v, node_out)
            pltpu.sync_copy(brow_v, brow_out)

    return sc_gather(am64, table, bondw)


def _lut_kernel(am_v, node_in, brow, spec, ew, eb, luthi, lutlo):
    rows = []
    for j in range(64):
        m = j >> 3
        if m <= 2:
            rows.append(spec[m : m + 1, :])
        else:
            rows.append(node_in[j : j + 1, :])
    node64 = jnp.concatenate(rows, axis=0)

    lane128 = lax.broadcasted_iota(jnp.int32, (64, 128), 1)
    lsel = am_v[...] % 128
    bc64 = jnp.sum(jnp.where(lane128 == lsel, brow[...], 0), axis=1, keepdims=True)

    r512 = lax.broadcasted_iota(jnp.int32, (NCOMBO, 64), 0)
    q64 = lax.broadcasted_iota(jnp.int32, (NCOMBO, 64), 1)
    ohe = (r512 // MAX_BOND == q64).astype(jnp.float32)
    node512 = jnp.dot(ohe, node64, preferred_element_type=jnp.float32)
    bc512 = jnp.dot(ohe, bc64.astype(jnp.float32), preferred_element_type=jnp.float32)
    bc512 = bc512.astype(jnp.int32)

    rowid = lax.broadcasted_iota(jnp.int32, (NCOMBO, MAX_BOND), 0)
    lane = lax.broadcasted_iota(jnp.int32, (NCOMBO, MAX_BOND), 1)
    bpos = rowid % MAX_BOND
    one_hot = jnp.where(lane == bpos, 1.0, jnp.where(lane < bc512, 0.0, -1.0))
    edge512 = jnp.dot(one_hot, ew[...], preferred_element_type=jnp.float32) + eb[...]

    nt = node512.T
    et = edge512.T
    nh = nt.astype(jnp.bfloat16)
    luthi[:NODE_DIM, :] = nh
    lutlo[:NODE_DIM, :] = (nt - nh.astype(jnp.float32)).astype(jnp.bfloat16)
    eh = et.astype(jnp.bfloat16)
    luthi[NODE_DIM:, :] = eh
    lutlo[NODE_DIM:, :] = (et - eh.astype(jnp.float32)).astype(jnp.bfloat16)


def _expand_kernel(idxt_ref, luthi, lutlo, out_ref, *, width, lanes):
    m = idxt_ref[0, :, :]
    a = idxt_ref[1, :, :]
    b = idxt_ref[2, :, :]
    c = (m * MAX_BOND + a) * MAX_BOND + b
    si = lax.broadcasted_iota(jnp.int32, (NCOMBO, lanes), 0)
    hi = luthi[...]
    lo = lutlo[...]
    for w in range(width):
        cw = c[w : w + 1, :]
        oh = (si == cw).astype(jnp.float32).astype(jnp.bfloat16)
        out_ref[w] = jnp.dot(hi, oh, preferred_element_type=jnp.float32) + jnp.dot(
            lo, oh, preferred_element_type=jnp.float32
        )


def kernel(idx, attached_motif_index_map, bonding_cnt, special_table, attached_table, edge_w, edge_b):
    nrows, width = idx.shape[:-1]
    am64 = attached_motif_index_map[:MAX_BOND, :MAX_BOND].reshape(64)
    npad = (-bonding_cnt.shape[0]) % 128
    bondw = jnp.pad(bonding_cnt, (0, npad)).reshape(-1, 128)

    node_in, brow = _sc_gather_call(am64, attached_table, bondw)

    luthi, lutlo = pl.pallas_call(
        _lut_kernel,
        out_shape=(
            jax.ShapeDtypeStruct((OUT_DIM, NCOMBO), jnp.bfloat16),
            jax.ShapeDtypeStruct((OUT_DIM, NCOMBO), jnp.bfloat16),
        ),
        in_specs=[
            pl.BlockSpec(memory_space=pltpu.VMEM),
            pl.BlockSpec(memory_space=pltpu.VMEM),
            pl.BlockSpec(memory_space=pltpu.VMEM),
            pl.BlockSpec(memory_space=pltpu.VMEM),
            pl.BlockSpec(memory_space=pltpu.VMEM),
            pl.BlockSpec(memory_space=pltpu.VMEM),
        ],
    )(am64.reshape(64, 1), node_in, brow, special_table, edge_w, edge_b.reshape(1, EDGE_DIM))

    lanes = LANES_PER_STEP
    assert nrows % lanes == 0

    idxt = jnp.transpose(idx, (2, 1, 0))

    outt = pl.pallas_call(
        functools.partial(_expand_kernel, width=width, lanes=lanes),
        grid=(nrows // lanes,),
        out_shape=jax.ShapeDtypeStruct((width, OUT_DIM, nrows), jnp.float32),
        in_specs=[
            pl.BlockSpec((3, width, lanes), lambda i: (0, 0, i)),
            pl.BlockSpec((OUT_DIM, NCOMBO), lambda i: (0, 0)),
            pl.BlockSpec((OUT_DIM, NCOMBO), lambda i: (0, 0)),
        ],
        out_specs=pl.BlockSpec((width, OUT_DIM, lanes), lambda i: (0, 0, i)),
        compiler_params=pltpu.CompilerParams(dimension_semantics=("parallel",)),
    )(idxt, luthi, lutlo)

    return jnp.transpose(outt, (2, 0, 1))


# speedup vs baseline: 1.0223x; 1.0223x over previous
import functools

import jax
import jax.numpy as jnp
from jax import lax
from jax.experimental import pallas as pl
from jax.experimental.pallas import tpu as pltpu
from jax.experimental.pallas import tpu_sc as plsc

NODE_DIM = 128
EDGE_DIM = 16
OUT_DIM = NODE_DIM + EDGE_DIM
MAX_BOND = 8
NCOMBO = 512
LANES_PER_STEP = 256


def _sc_gather_call(am64, table, bondw):
    mesh = plsc.VectorSubcoreMesh(core_axis_name="c", subcore_axis_name="s")

    @functools.partial(
        pl.kernel,
        mesh=mesh,
        out_type=(
            jax.ShapeDtypeStruct((64, NODE_DIM), jnp.float32),
            jax.ShapeDtypeStruct((64, 128), jnp.int32),
        ),
        scratch_types=[
            pltpu.VMEM((64,), jnp.int32),
            pltpu.VMEM((64,), jnp.int32),
            pltpu.VMEM((64, NODE_DIM), jnp.float32),
            pltpu.VMEM((64, 128), jnp.int32),
            pltpu.SemaphoreType.DMA,
            pltpu.SemaphoreType.DMA,
        ],
    )
    def sc_gather(am_hbm, table_hbm, bondw_hbm, node_out, brow_out, idx_v, idx7_v, rows_v, brow_v, sem1, sem2):
        cid = lax.axis_index("c")
        sid = lax.axis_index("s")
        wid = sid * 2 + cid

        @pl.when(wid == 0)
        def _():
            pltpu.sync_copy(am_hbm, idx_v)
            cp1 = pltpu.async_copy(table_hbm.at[idx_v], rows_v, sem1)
            cp1.wait()
            pltpu.sync_copy(rows_v, node_out)

        @pl.when(wid == 1)
        def _():
            pltpu.sync_copy(am_hbm, idx_v)
            for k in range(4):
                idx7_v[pl.ds(k * 16, 16)] = idx_v[pl.ds(k * 16, 16)] >> 7
            cp2 = pltpu.async_copy(bondw_hbm.at[idx7_v], brow_v, sem2)
            cp2.wait()
            pltpu.sync_copy(brow_v, brow_out)

    return sc_gather(am64, table, bondw)


def _build_lut(am_v, node_in, brow, spec, ew, eb, luthi, lutlo):
    rows = []
    for j in range(64):
        m = j >> 3
        if m <= 2:
            rows.append(spec[m : m + 1, :])
        else:
            rows.append(node_in[j : j + 1, :])
    node64 = jnp.concatenate(rows, axis=0)

    lane128 = lax.broadcasted_iota(jnp.int32, (64, 128), 1)
    lsel = am_v[...] % 128
    bc64 = jnp.sum(jnp.where(lane128 == lsel, brow[...], 0), axis=1, keepdims=True)

    r512 = lax.broadcasted_iota(jnp.int32, (NCOMBO, 64), 0)
    q64 = lax.broadcasted_iota(jnp.int32, (NCOMBO, 64), 1)
    ohe = (r512 // MAX_BOND == q64).astype(jnp.float32)
    node512 = jnp.dot(ohe, node64, preferred_element_type=jnp.float32)
    bc512 = jnp.dot(ohe, bc64.astype(jnp.float32), preferred_element_type=jnp.float32)
    bc512 = bc512.astype(jnp.int32)

    rowid = lax.broadcasted_iota(jnp.int32, (NCOMBO, MAX_BOND), 0)
    lane = lax.broadcasted_iota(jnp.int32, (NCOMBO, MAX_BOND), 1)
    bpos = rowid % MAX_BOND
    one_hot = jnp.where(lane == bpos, 1.0, jnp.where(lane < bc512, 0.0, -1.0))
    edge512 = jnp.dot(one_hot, ew[...], preferred_element_type=jnp.float32) + eb[...]

    nt = node512.T
    et = edge512.T
    nh = nt.astype(jnp.bfloat16)
    luthi[:NODE_DIM, :] = nh
    lutlo[:NODE_DIM, :] = (nt - nh.astype(jnp.float32)).astype(jnp.bfloat16)
    eh = et.astype(jnp.bfloat16)
    luthi[NODE_DIM:, :] = eh
    lutlo[NODE_DIM:, :] = (et - eh.astype(jnp.float32)).astype(jnp.bfloat16)


def _expand_kernel(idxt_ref, am_v, node_in, brow, spec, ew, eb, out_ref, luthi, lutlo, *, width, lanes):
    @pl.when(pl.program_id(0) == 0)
    def _():
        _build_lut(am_v, node_in, brow, spec, ew, eb, luthi, lutlo)

    m = idxt_ref[0, :, :]
    a = idxt_ref[1, :, :]
    b = idxt_ref[2, :, :]
    c = (m * MAX_BOND + a) * MAX_BOND + b
    si = lax.broadcasted_iota(jnp.int32, (NCOMBO, lanes), 0)
    hi = luthi[...]
    lo = lutlo[...]
    for w in range(width):
        cw = c[w : w + 1, :]
        oh = (si == cw).astype(jnp.float32).astype(jnp.bfloat16)
        out_ref[w] = jnp.dot(hi, oh, preferred_element_type=jnp.float32) + jnp.dot(
            lo, oh, preferred_element_type=jnp.float32
        )


def kernel(idx, attached_motif_index_map, bonding_cnt, special_table, attached_table, edge_w, edge_b):
    nrows, width = idx.shape[:-1]
    am64 = attached_motif_index_map[:MAX_BOND, :MAX_BOND].reshape(64)
    npad = (-bonding_cnt.shape[0]) % 128
    bondw = jnp.pad(bonding_cnt, (0, npad)).reshape(-1, 128)

    node_in, brow = _sc_gather_call(am64, attached_table, bondw)

    lanes = LANES_PER_STEP
    assert nrows % lanes == 0

    idxt = jnp.transpose(idx, (2, 1, 0))

    outt = pl.pallas_call(
        functools.partial(_expand_kernel, width=width, lanes=lanes),
        grid=(nrows // lanes,),
        out_shape=jax.ShapeDtypeStruct((width, OUT_DIM, nrows), jnp.float32),
        in_specs=[
            pl.BlockSpec((3, width, lanes), lambda i: (0, 0, i)),
            pl.BlockSpec((64, 1), lambda i: (0, 0)),
            pl.BlockSpec((64, NODE_DIM), lambda i: (0, 0)),
            pl.BlockSpec((64, 128), lambda i: (0, 0)),
            pl.BlockSpec((3, NODE_DIM), lambda i: (0, 0)),
            pl.BlockSpec((MAX_BOND, EDGE_DIM), lambda i: (0, 0)),
            pl.BlockSpec((1, EDGE_DIM), lambda i: (0, 0)),
        ],
        out_specs=pl.BlockSpec((width, OUT_DIM, lanes), lambda i: (0, 0, i)),
        scratch_shapes=[
            pltpu.VMEM((OUT_DIM, NCOMBO), jnp.bfloat16),
            pltpu.VMEM((OUT_DIM, NCOMBO), jnp.bfloat16),
        ],
    )(idxt, am64.reshape(64, 1), node_in, brow, special_table, edge_w, edge_b.reshape(1, EDGE_DIM))

    return jnp.transpose(outt, (2, 0, 1))


# speedup vs baseline: 1.3515x; 1.3220x over previous
import functools

import jax
import jax.numpy as jnp
from jax import lax
from jax.experimental import pallas as pl
from jax.experimental.pallas import tpu as pltpu
from jax.experimental.pallas import tpu_sc as plsc

NODE_DIM = 128
EDGE_DIM = 16
OUT_DIM = NODE_DIM + EDGE_DIM
MAX_BOND = 8
LANES_PER_STEP = 256


def _sc_gather_call(am64, table, bondw):
    mesh = plsc.VectorSubcoreMesh(core_axis_name="c", subcore_axis_name="s")

    @functools.partial(
        pl.kernel,
        mesh=mesh,
        out_type=(
            jax.ShapeDtypeStruct((64, NODE_DIM), jnp.float32),
            jax.ShapeDtypeStruct((64, 128), jnp.int32),
        ),
        scratch_types=[
            pltpu.VMEM((64,), jnp.int32),
            pltpu.VMEM((64,), jnp.int32),
            pltpu.VMEM((64, NODE_DIM), jnp.float32),
            pltpu.VMEM((64, 128), jnp.int32),
            pltpu.SemaphoreType.DMA,
            pltpu.SemaphoreType.DMA,
        ],
    )
    def sc_gather(am_hbm, table_hbm, bondw_hbm, node_out, brow_out, idx_v, idx7_v, rows_v, brow_v, sem1, sem2):
        cid = lax.axis_index("c")
        sid = lax.axis_index("s")
        wid = sid * 2 + cid

        @pl.when(wid == 0)
        def _():
            pltpu.sync_copy(am_hbm, idx_v)
            cp1 = pltpu.async_copy(table_hbm.at[idx_v], rows_v, sem1)
            cp1.wait()
            pltpu.sync_copy(rows_v, node_out)

        @pl.when(wid == 1)
        def _():
            pltpu.sync_copy(am_hbm, idx_v)
            for k in range(4):
                idx7_v[pl.ds(k * 16, 16)] = idx_v[pl.ds(k * 16, 16)] >> 7
            cp2 = pltpu.async_copy(bondw_hbm.at[idx7_v], brow_v, sem2)
            cp2.wait()
            pltpu.sync_copy(brow_v, brow_out)

    return sc_gather(am64, table, bondw)


def _build_lut(am_v, node_in, brow, spec, ew, eb, tabhi, tablo, words):
    rows = []
    for j in range(64):
        m = j >> 3
        if m <= 2:
            rows.append(spec[m : m + 1, :])
        else:
            rows.append(node_in[j : j + 1, :])
    node64 = jnp.concatenate(rows, axis=0)

    lane128 = lax.broadcasted_iota(jnp.int32, (64, 128), 1)
    lsel = am_v[...] % 128
    bc64 = jnp.sum(jnp.where(lane128 == lsel, brow[...], 0), axis=1, keepdims=True)

    lane8 = lax.broadcasted_iota(jnp.int32, (64, MAX_BOND), 1)
    mask_ge = (lane8 >= bc64).astype(jnp.float32)
    s = jnp.dot(mask_ge, ew[...], preferred_element_type=jnp.float32)
    t = eb[...] - s

    nt = node64.T
    tt = t.T
    nh = nt.astype(jnp.bfloat16)
    th = tt.astype(jnp.bfloat16)
    tabhi[:NODE_DIM, :] = nh
    tabhi[NODE_DIM:, :] = th
    tablo[:NODE_DIM, :] = (nt - nh.astype(jnp.float32)).astype(jnp.bfloat16)
    tablo[NODE_DIM:, :] = (tt - th.astype(jnp.float32)).astype(jnp.bfloat16)

    i0 = lax.broadcasted_iota(jnp.int32, (64, MAX_BOND), 0)
    i1 = lax.broadcasted_iota(jnp.int32, (64, MAX_BOND), 1)
    pw = jnp.where(i0 >> 3 == i1, (1 << (3 * (i0 % MAX_BOND))).astype(jnp.float32), 0.0)
    bcm1t = (bc64 - 1).astype(jnp.float32).T
    wf = jnp.dot(bcm1t, pw, preferred_element_type=jnp.float32)
    words[...] = wf.astype(jnp.int32)


def _expand_kernel(idxt_ref, am_v, node_in, brow, spec, ew, eb, ewht, ewlt, out_ref, tabhi, tablo, words, *, width, lanes):
    @pl.when(pl.program_id(0) == 0)
    def _():
        _build_lut(am_v, node_in, brow, spec, ew, eb, tabhi, tablo, words)

    m = idxt_ref[0, :, :]
    a = idxt_ref[1, :, :]
    b = idxt_ref[2, :, :]
    c2 = m * MAX_BOND + a

    k = c2 >> 3
    sh = (c2 & 7) * 3
    word = jnp.zeros((width, lanes), jnp.int32)
    for j in range(MAX_BOND):
        wj = words[0:1, j : j + 1]
        word = word + jnp.where(k == j, wj, 0)
    bcall = ((word >> sh) & 7) + 1

    si64 = lax.broadcasted_iota(jnp.int32, (64, lanes), 0)
    si8 = lax.broadcasted_iota(jnp.int32, (MAX_BOND, lanes), 0)
    hi = tabhi[...]
    lo = tablo[...]
    ewh = ewht[...]
    ewl = ewlt[...]
    for w in range(width):
        c2w = c2[w : w + 1, :]
        bw = b[w : w + 1, :]
        oh = (si64 == c2w).astype(jnp.bfloat16)
        full = jnp.dot(hi, oh, preferred_element_type=jnp.float32) + jnp.dot(
            lo, oh, preferred_element_type=jnp.float32
        )
        two = jnp.where(bw >= bcall[w : w + 1, :], 2.0, 1.0)
        oh8 = jnp.where(si8 == bw, two, 0.0).astype(jnp.bfloat16)
        ewterm = jnp.dot(ewh, oh8, preferred_element_type=jnp.float32) + jnp.dot(
            ewl, oh8, preferred_element_type=jnp.float32
        )
        out_ref[w, :NODE_DIM, :] = full[:NODE_DIM, :]
        out_ref[w, NODE_DIM:, :] = full[NODE_DIM:, :] + ewterm


def kernel(idx, attached_motif_index_map, bonding_cnt, special_table, attached_table, edge_w, edge_b):
    nrows, width = idx.shape[:-1]
    am64 = attached_motif_index_map[:MAX_BOND, :MAX_BOND].reshape(64)
    npad = (-bonding_cnt.shape[0]) % 128
    bondw = jnp.pad(bonding_cnt, (0, npad)).reshape(-1, 128)

    node_in, brow = _sc_gather_call(am64, attached_table, bondw)

    ewt = edge_w.T
    ewht = ewt.astype(jnp.bfloat16)
    ewlt = (ewt - ewht.astype(jnp.float32)).astype(jnp.bfloat16)

    lanes = LANES_PER_STEP
    assert nrows % lanes == 0

    idxt = jnp.transpose(idx, (2, 1, 0))

    outt = pl.pallas_call(
        functools.partial(_expand_kernel, width=width, lanes=lanes),
        grid=(nrows // lanes,),
        out_shape=jax.ShapeDtypeStruct((width, OUT_DIM, nrows), jnp.float32),
        in_specs=[
            pl.BlockSpec((3, width, lanes), lambda i: (0, 0, i)),
            pl.BlockSpec((64, 1), lambda i: (0, 0)),
            pl.BlockSpec((64, NODE_DIM), lambda i: (0, 0)),
            pl.BlockSpec((64, 128), lambda i: (0, 0)),
            pl.BlockSpec((3, NODE_DIM), lambda i: (0, 0)),
            pl.BlockSpec((MAX_BOND, EDGE_DIM), lambda i: (0, 0)),
            pl.BlockSpec((1, EDGE_DIM), lambda i: (0, 0)),
            pl.BlockSpec((EDGE_DIM, MAX_BOND), lambda i: (0, 0)),
            pl.BlockSpec((EDGE_DIM, MAX_BOND), lambda i: (0, 0)),
        ],
        out_specs=pl.BlockSpec((width, OUT_DIM, lanes), lambda i: (0, 0, i)),
        scratch_shapes=[
            pltpu.VMEM((OUT_DIM, 64), jnp.bfloat16),
            pltpu.VMEM((OUT_DIM, 64), jnp.bfloat16),
            pltpu.VMEM((1, MAX_BOND), jnp.int32),
        ],
    )(idxt, am64.reshape(64, 1), node_in, brow, special_table, edge_w, edge_b.reshape(1, EDGE_DIM), ewht, ewlt)

    return jnp.transpose(outt, (2, 0, 1))


# speedup vs baseline: 1.3949x; 1.0321x over previous
import functools

import jax
import jax.numpy as jnp
from jax import lax
from jax.experimental import pallas as pl
from jax.experimental.pallas import tpu as pltpu
from jax.experimental.pallas import tpu_sc as plsc

NODE_DIM = 128
EDGE_DIM = 16
OUT_DIM = NODE_DIM + EDGE_DIM
MAX_BOND = 8
LANES_PER_STEP = 512


def _sc_gather_call(am64, table, bondw):
    mesh = plsc.VectorSubcoreMesh(core_axis_name="c", subcore_axis_name="s")

    @functools.partial(
        pl.kernel,
        mesh=mesh,
        out_type=(
            jax.ShapeDtypeStruct((64, NODE_DIM), jnp.float32),
            jax.ShapeDtypeStruct((64, 128), jnp.int32),
        ),
        scratch_types=[
            pltpu.VMEM((64,), jnp.int32),
            pltpu.VMEM((64,), jnp.int32),
            pltpu.VMEM((64, NODE_DIM), jnp.float32),
            pltpu.VMEM((64, 128), jnp.int32),
            pltpu.SemaphoreType.DMA,
            pltpu.SemaphoreType.DMA,
        ],
    )
    def sc_gather(am_hbm, table_hbm, bondw_hbm, node_out, brow_out, idx_v, idx7_v, rows_v, brow_v, sem1, sem2):
        cid = lax.axis_index("c")
        sid = lax.axis_index("s")
        wid = sid * 2 + cid

        @pl.when(wid == 0)
        def _():
            pltpu.sync_copy(am_hbm, idx_v)
            cp1 = pltpu.async_copy(table_hbm.at[idx_v], rows_v, sem1)
            cp1.wait()
            pltpu.sync_copy(rows_v, node_out)

        @pl.when(wid == 1)
        def _():
            pltpu.sync_copy(am_hbm, idx_v)
            for k in range(4):
                idx7_v[pl.ds(k * 16, 16)] = idx_v[pl.ds(k * 16, 16)] >> 7
            cp2 = pltpu.async_copy(bondw_hbm.at[idx7_v], brow_v, sem2)
            cp2.wait()
            pltpu.sync_copy(brow_v, brow_out)

    return sc_gather(am64, table, bondw)


def _build_lut(am_v, node_in, brow, spec, ew, eb, tabhi, tablo, words):
    rows = []
    for j in range(64):
        m = j >> 3
        if m <= 2:
            rows.append(spec[m : m + 1, :])
        else:
            rows.append(node_in[j : j + 1, :])
    node64 = jnp.concatenate(rows, axis=0)

    lane128 = lax.broadcasted_iota(jnp.int32, (64, 128), 1)
    lsel = am_v[...] % 128
    bc64 = jnp.sum(jnp.where(lane128 == lsel, brow[...], 0), axis=1, keepdims=True)

    lane8 = lax.broadcasted_iota(jnp.int32, (64, MAX_BOND), 1)
    mask_ge = (lane8 >= bc64).astype(jnp.float32)
    s = jnp.dot(mask_ge, ew[...], preferred_element_type=jnp.float32)
    t = eb[...] - s

    nt = node64.T
    tt = t.T
    nh = nt.astype(jnp.bfloat16)
    th = tt.astype(jnp.bfloat16)
    tabhi[:NODE_DIM, :] = nh
    tabhi[NODE_DIM:, :] = th
    tablo[:NODE_DIM, :] = (nt - nh.astype(jnp.float32)).astype(jnp.bfloat16)
    tablo[NODE_DIM:, :] = (tt - th.astype(jnp.float32)).astype(jnp.bfloat16)

    i0 = lax.broadcasted_iota(jnp.int32, (64, MAX_BOND), 0)
    i1 = lax.broadcasted_iota(jnp.int32, (64, MAX_BOND), 1)
    pw = jnp.where(i0 >> 3 == i1, (1 << (3 * (i0 % MAX_BOND))).astype(jnp.float32), 0.0)
    bcm1t = (bc64 - 1).astype(jnp.float32).T
    wf = jnp.dot(bcm1t, pw, preferred_element_type=jnp.float32)
    words[...] = wf.astype(jnp.int32)


def _expand_kernel(idxt_ref, am_v, node_in, brow, spec, ew, eb, ewht, ewlt, out_ref, tabhi, tablo, words, *, width, lanes):
    @pl.when(pl.program_id(0) == 0)
    def _():
        _build_lut(am_v, node_in, brow, spec, ew, eb, tabhi, tablo, words)

    m = idxt_ref[0, :, :]
    a = idxt_ref[1, :, :]
    b = idxt_ref[2, :, :]
    c2 = m * MAX_BOND + a

    k = c2 >> 3
    sh = (c2 & 7) * 3
    word = jnp.zeros((width, lanes), jnp.int32)
    for j in range(MAX_BOND):
        wj = words[0:1, j : j + 1]
        word = word + jnp.where(k == j, wj, 0)
    bcall = ((word >> sh) & 7) + 1

    si64 = lax.broadcasted_iota(jnp.int32, (64, lanes), 0)
    si8 = lax.broadcasted_iota(jnp.int32, (MAX_BOND, lanes), 0)
    hi = tabhi[...]
    lo = tablo[...]
    ewh = ewht[...]
    ewl = ewlt[...]
    for w in range(width):
        c2w = c2[w : w + 1, :]
        bw = b[w : w + 1, :]
        oh = (si64 == c2w).astype(jnp.bfloat16)
        full = jnp.dot(hi, oh, preferred_element_type=jnp.float32) + jnp.dot(
            lo, oh, preferred_element_type=jnp.float32
        )
        two = jnp.where(bw >= bcall[w : w + 1, :], 2.0, 1.0)
        oh8 = jnp.where(si8 == bw, two, 0.0).astype(jnp.bfloat16)
        ewterm = jnp.dot(ewh, oh8, preferred_element_type=jnp.float32) + jnp.dot(
            ewl, oh8, preferred_element_type=jnp.float32
        )
        out_ref[w, :NODE_DIM, :] = full[:NODE_DIM, :]
        out_ref[w, NODE_DIM:, :] = full[NODE_DIM:, :] + ewterm


def kernel(idx, attached_motif_index_map, bonding_cnt, special_table, attached_table, edge_w, edge_b):
    nrows, width = idx.shape[:-1]
    am64 = attached_motif_index_map[:MAX_BOND, :MAX_BOND].reshape(64)
    npad = (-bonding_cnt.shape[0]) % 128
    bondw = jnp.pad(bonding_cnt, (0, npad)).reshape(-1, 128)

    node_in, brow = _sc_gather_call(am64, attached_table, bondw)

    ewt = edge_w.T
    ewht = ewt.astype(jnp.bfloat16)
    ewlt = (ewt - ewht.astype(jnp.float32)).astype(jnp.bfloat16)

    lanes = LANES_PER_STEP
    assert nrows % lanes == 0

    idxt = jnp.transpose(idx, (2, 1, 0))

    outt = pl.pallas_call(
        functools.partial(_expand_kernel, width=width, lanes=lanes),
        grid=(nrows // lanes,),
        out_shape=jax.ShapeDtypeStruct((width, OUT_DIM, nrows), jnp.float32),
        in_specs=[
            pl.BlockSpec((3, width, lanes), lambda i: (0, 0, i)),
            pl.BlockSpec((64, 1), lambda i: (0, 0)),
            pl.BlockSpec((64, NODE_DIM), lambda i: (0, 0)),
            pl.BlockSpec((64, 128), lambda i: (0, 0)),
            pl.BlockSpec((3, NODE_DIM), lambda i: (0, 0)),
            pl.BlockSpec((MAX_BOND, EDGE_DIM), lambda i: (0, 0)),
            pl.BlockSpec((1, EDGE_DIM), lambda i: (0, 0)),
            pl.BlockSpec((EDGE_DIM, MAX_BOND), lambda i: (0, 0)),
            pl.BlockSpec((EDGE_DIM, MAX_BOND), lambda i: (0, 0)),
        ],
        out_specs=pl.BlockSpec((width, OUT_DIM, lanes), lambda i: (0, 0, i)),
        scratch_shapes=[
            pltpu.VMEM((OUT_DIM, 64), jnp.bfloat16),
            pltpu.VMEM((OUT_DIM, 64), jnp.bfloat16),
            pltpu.VMEM((1, MAX_BOND), jnp.int32),
        ],
    )(idxt, am64.reshape(64, 1), node_in, brow, special_table, edge_w, edge_b.reshape(1, EDGE_DIM), ewht, ewlt)

    return jnp.transpose(outt, (2, 0, 1))


# speedup vs baseline: 1.5136x; 1.0851x over previous
import functools

import jax
import jax.numpy as jnp
from jax import lax
from jax.experimental import pallas as pl
from jax.experimental.pallas import tpu as pltpu
from jax.experimental.pallas import tpu_sc as plsc

NODE_DIM = 128
EDGE_DIM = 16
OUT_DIM = NODE_DIM + EDGE_DIM
MAX_BOND = 8
LANES_PER_STEP = 512


def _sc_gather_call(am64, table, bondw):
    mesh = plsc.VectorSubcoreMesh(core_axis_name="c", subcore_axis_name="s")

    @functools.partial(
        pl.kernel,
        mesh=mesh,
        out_type=(
            jax.ShapeDtypeStruct((64, NODE_DIM), jnp.float32),
            jax.ShapeDtypeStruct((64, 128), jnp.int32),
        ),
        scratch_types=[
            pltpu.VMEM((64,), jnp.int32),
            pltpu.VMEM((64,), jnp.int32),
            pltpu.VMEM((64, NODE_DIM), jnp.float32),
            pltpu.VMEM((64, 128), jnp.int32),
            pltpu.SemaphoreType.DMA,
            pltpu.SemaphoreType.DMA,
        ],
    )
    def sc_gather(am_hbm, table_hbm, bondw_hbm, node_out, brow_out, idx_v, idx7_v, rows_v, brow_v, sem1, sem2):
        cid = lax.axis_index("c")
        sid = lax.axis_index("s")
        wid = sid * 2 + cid

        @pl.when(wid == 0)
        def _():
            pltpu.sync_copy(am_hbm, idx_v)
            cp1 = pltpu.async_copy(table_hbm.at[idx_v], rows_v, sem1)
            cp1.wait()
            pltpu.sync_copy(rows_v, node_out)

        @pl.when(wid == 1)
        def _():
            pltpu.sync_copy(am_hbm, idx_v)
            for k in range(4):
                idx7_v[pl.ds(k * 16, 16)] = idx_v[pl.ds(k * 16, 16)] >> 7
            cp2 = pltpu.async_copy(bondw_hbm.at[idx7_v], brow_v, sem2)
            cp2.wait()
            pltpu.sync_copy(brow_v, brow_out)

    return sc_gather(am64, table, bondw)


def _build_lut(am_v, node_in, brow, spec, ew, eb, tab, words):
    rows = []
    for j in range(64):
        m = j >> 3
        if m <= 2:
            rows.append(spec[m : m + 1, :])
        else:
            rows.append(node_in[j : j + 1, :])
    node64 = jnp.concatenate(rows, axis=0)

    lane128 = lax.broadcasted_iota(jnp.int32, (64, 128), 1)
    lsel = am_v[...] % 128
    bc64 = jnp.sum(jnp.where(lane128 == lsel, brow[...], 0), axis=1, keepdims=True)

    lane8 = lax.broadcasted_iota(jnp.int32, (64, MAX_BOND), 1)
    mask_ge = (lane8 >= bc64).astype(jnp.float32)
    s = jnp.dot(mask_ge, ew[...], preferred_element_type=jnp.float32)
    t = eb[...] - s

    nt = node64.T
    tt = t.T
    nh = nt.astype(jnp.bfloat16)
    th = tt.astype(jnp.bfloat16)
    tab[:NODE_DIM, 0:64] = nh
    tab[NODE_DIM:, 0:64] = th
    tab[:NODE_DIM, 64:128] = (nt - nh.astype(jnp.float32)).astype(jnp.bfloat16)
    tab[NODE_DIM:, 64:128] = (tt - th.astype(jnp.float32)).astype(jnp.bfloat16)

    i0 = lax.broadcasted_iota(jnp.int32, (64, MAX_BOND), 0)
    i1 = lax.broadcasted_iota(jnp.int32, (64, MAX_BOND), 1)
    pw = jnp.where(i0 >> 3 == i1, (1 << (3 * (i0 % MAX_BOND))).astype(jnp.float32), 0.0)
    bcm1t = (bc64 - 1).astype(jnp.float32).T
    wf = jnp.dot(bcm1t, pw, preferred_element_type=jnp.float32)
    words[...] = wf.astype(jnp.int32)


def _expand_kernel(idxt_ref, am_v, node_in, brow, spec, ew, eb, ewcat, out_ref, tab, words, *, width, lanes):
    @pl.when(pl.program_id(0) == 0)
    def _():
        _build_lut(am_v, node_in, brow, spec, ew, eb, tab, words)

    m = idxt_ref[0, :, :]
    a = idxt_ref[1, :, :]
    b = idxt_ref[2, :, :]
    c2 = m * MAX_BOND + a

    k = c2 >> 3
    sh = (c2 & 7) * 3
    word = jnp.zeros((width, lanes), jnp.int32)
    for j in range(MAX_BOND):
        wj = words[0:1, j : j + 1]
        word = word + jnp.where(k == j, wj, 0)
    bcall = ((word >> sh) & 7) + 1

    si128 = lax.broadcasted_iota(jnp.int32, (128, lanes), 0)
    si16 = lax.broadcasted_iota(jnp.int32, (2 * MAX_BOND, lanes), 0)
    tabv = tab[...]
    ewc = ewcat[...]
    for w in range(width):
        c2w = c2[w : w + 1, :]
        bw = b[w : w + 1, :]
        oh = ((si128 & 63) == c2w).astype(jnp.bfloat16)
        full = jnp.dot(tabv, oh, preferred_element_type=jnp.float32)
        two = jnp.where(bw >= bcall[w : w + 1, :], 2.0, 1.0)
        oh8 = jnp.where((si16 & 7) == bw, two, 0.0).astype(jnp.bfloat16)
        ewterm = jnp.dot(ewc, oh8, preferred_element_type=jnp.float32)
        out_ref[w, :NODE_DIM, :] = full[:NODE_DIM, :]
        out_ref[w, NODE_DIM:, :] = full[NODE_DIM:, :] + ewterm


def kernel(idx, attached_motif_index_map, bonding_cnt, special_table, attached_table, edge_w, edge_b):
    nrows, width = idx.shape[:-1]
    am64 = attached_motif_index_map[:MAX_BOND, :MAX_BOND].reshape(64)
    npad = (-bonding_cnt.shape[0]) % 128
    bondw = jnp.pad(bonding_cnt, (0, npad)).reshape(-1, 128)

    node_in, brow = _sc_gather_call(am64, attached_table, bondw)

    ewt = edge_w.T
    ewht = ewt.astype(jnp.bfloat16)
    ewlt = (ewt - ewht.astype(jnp.float32)).astype(jnp.bfloat16)
    ewcat = jnp.concatenate([ewht, ewlt], axis=1)

    lanes = LANES_PER_STEP
    assert nrows % lanes == 0

    idxt = jnp.transpose(idx, (2, 1, 0))

    outt = pl.pallas_call(
        functools.partial(_expand_kernel, width=width, lanes=lanes),
        grid=(nrows // lanes,),
        out_shape=jax.ShapeDtypeStruct((width, OUT_DIM, nrows), jnp.float32),
        in_specs=[
            pl.BlockSpec((3, width, lanes), lambda i: (0, 0, i)),
            pl.BlockSpec((64, 1), lambda i: (0, 0)),
            pl.BlockSpec((64, NODE_DIM), lambda i: (0, 0)),
            pl.BlockSpec((64, 128), lambda i: (0, 0)),
            pl.BlockSpec((3, NODE_DIM), lambda i: (0, 0)),
            pl.BlockSpec((MAX_BOND, EDGE_DIM), lambda i: (0, 0)),
            pl.BlockSpec((1, EDGE_DIM), lambda i: (0, 0)),
            pl.BlockSpec((EDGE_DIM, 2 * MAX_BOND), lambda i: (0, 0)),
        ],
        out_specs=pl.BlockSpec((width, OUT_DIM, lanes), lambda i: (0, 0, i)),
        scratch_shapes=[
            pltpu.VMEM((OUT_DIM, 128), jnp.bfloat16),
            pltpu.VMEM((1, MAX_BOND), jnp.int32),
        ],
    )(idxt, am64.reshape(64, 1), node_in, brow, special_table, edge_w, edge_b.reshape(1, EDGE_DIM), ewcat)

    return jnp.transpose(outt, (2, 0, 1))
